# baseline (speedup 1.0000x reference)
import jax
import jax.numpy as jnp
from jax.experimental import pallas as pl
from jax.experimental.pallas import tpu as pltpu


def _wsad_body(x1_ref, x2_ref, x3_ref, x4_ref, wet_ref, be_ref, wc1t_ref,
               wc2t_ref, wstack_ref, bt_ref, bcls_ref, out_ref):
    xrefs = (x1_ref, x2_ref, x3_ref, x4_ref)
    n = x1_ref.shape[1]
    t = x1_ref.shape[2]
    dq = x1_ref.shape[3]
    tt = n * t

    parts = []
    for q, xr in enumerate(xrefs):
        xb = xr[0].reshape(tt, dq).astype(jnp.bfloat16)
        parts.append(jax.lax.dot_general(
            wet_ref[:, q * dq:(q + 1) * dq], xb, (((1,), (1,)), ((), ())),
            preferred_element_type=jnp.float32))
    hT = (parts[0] + parts[1]) + (parts[2] + parts[3])
    hT = jnp.maximum(hT + be_ref[...], 0.0)

    u = jax.lax.dot_general(
        wc1t_ref[...], hT, (((1,), (0,)), ((), ())),
        preferred_element_type=jnp.float32)
    z = jax.lax.dot_general(
        wstack_ref[...], hT, (((1,), (0,)), ((), ())),
        preferred_element_type=jnp.float32)
    tatt = jax.nn.sigmoid(z[0:1, :] + bt_ref[0, 0])

    ones8 = jnp.full((t, 8), 1.0 / t, jnp.float32)
    he_sum = None
    se_sum = None
    ss_sum = None
    for c in range(n):
        sl = slice(c * t, (c + 1) * t)
        g = jax.lax.dot_general(
            u[:, sl], ones8, (((1,), (0,)), ((), ())),
            preferred_element_type=jnp.float32)
        c1 = jnp.maximum(g, 0.0)
        c8 = jax.lax.dot_general(
            wc2t_ref[...], c1, (((1,), (0,)), ((), ())),
            preferred_element_type=jnp.float32)
        catten = jax.nn.sigmoid(c8[:, 0:1])
        he_c = hT[:, sl] * catten
        he_sum = he_c if he_sum is None else he_sum + he_c
        zhe_c = jax.lax.dot_general(
            wstack_ref[...], he_c, (((1,), (0,)), ((), ())),
            preferred_element_type=jnp.float32)
        se_c = jax.nn.sigmoid(zhe_c[1:2, :] + bcls_ref[0, 0])
        ss_c = jax.nn.sigmoid(z[1:2, sl] - zhe_c[1:2, :] + bcls_ref[0, 0])
        se_sum = se_c if se_sum is None else se_sum + se_c
        ss_sum = ss_c if ss_sum is None else ss_sum + ss_c

    te_sum = tatt[:, 0:t]
    for c in range(1, n):
        te_sum = te_sum + tatt[:, c * t:(c + 1) * t]

    k = t // 16 + 1
    inv_n = 1.0 / n
    score_e_m = se_sum * inv_n
    score_s_m = ss_sum * inv_n
    te = te_sum * inv_n
    ts = 1.0 - te

    def softmax_row(v):
        e = jnp.exp(v - jnp.max(v))
        return e / jnp.sum(e)

    we_ = softmax_row(te)
    ws_ = softmax_row(ts)
    bag_ee = jnp.sum(score_e_m * we_)
    bag_es = jnp.sum(score_e_m * ws_)
    bag_se = jnp.sum(score_s_m * we_)
    bag_ss = jnp.sum(score_s_m * ws_)

    sc_scaled = score_e_m * te
    fm = he_sum * inv_n
    magsq = jnp.sum(fm * fm, axis=0, keepdims=True)
    rm = jnp.sqrt(magsq) * sc_scaled

    ir = jax.lax.broadcasted_iota(jnp.int32, (t, t), 0)
    ic = jax.lax.broadcasted_iota(jnp.int32, (t, t), 1)
    idm = (ir == ic).astype(jnp.float32)
    rm_col = jnp.transpose(rm)
    beats = (rm_col > rm) | ((rm_col == rm) & (ir < ic))
    m = beats.astype(jnp.float32)
    ones8r = jnp.ones((8, t), jnp.float32)
    rank8 = jax.lax.dot_general(
        ones8r, m, (((1,), (0,)), ((), ())),
        preferred_element_type=jnp.float32)
    rank_col = jax.lax.dot_general(
        idm, rank8, (((1,), (1,)), ((), ())),
        preferred_element_type=jnp.float32)[:, 0:1]
    ranks128 = jax.lax.broadcasted_iota(
        jnp.int32, (t, 128), 1).astype(jnp.float32)
    onehot = (rank_col == ranks128).astype(jnp.float32)
    v_sc = jax.lax.dot_general(
        sc_scaled, onehot, (((1,), (0,)), ((), ())),
        preferred_element_type=jnp.float32)
    v_ref = jax.lax.dot_general(
        rm, onehot, (((1,), (0,)), ((), ())),
        preferred_element_type=jnp.float32)

    row = jnp.concatenate([
        v_sc[:, :k], v_ref[:, :k],
        jnp.stack([bag_ee, bag_es, bag_se, bag_ss])[None, :],
        jnp.zeros((1, 128 - (2 * k + 4)), jnp.float32),
    ], axis=1)
    out_ref[0] = row


@jax.jit
def kernel(x, W_enh, b_enh, Wc1, Wc2, Wt, bt, Wcls, bcls):
    b, n, t, d = x.shape
    dh = W_enh.shape[1]
    dm = Wc1.shape[1]
    k = t // 16 + 1

    wstack = jnp.concatenate(
        [Wt.reshape(1, dh), Wcls.reshape(1, dh),
         jnp.zeros((6, dh), jnp.float32)], axis=0)

    out = pl.pallas_call(
        _wsad_body,
        grid=(b,),
        in_specs=[
            pl.BlockSpec((1, n, t, d // 4), lambda i: (i, 0, 0, 0)),
            pl.BlockSpec((1, n, t, d // 4), lambda i: (i, 0, 0, 1)),
            pl.BlockSpec((1, n, t, d // 4), lambda i: (i, 0, 0, 2)),
            pl.BlockSpec((1, n, t, d // 4), lambda i: (i, 0, 0, 3)),
            pl.BlockSpec((dh, d), lambda i: (0, 0)),
            pl.BlockSpec((dh, 1), lambda i: (0, 0)),
            pl.BlockSpec((dm, dh), lambda i: (0, 0)),
            pl.BlockSpec((dh, dm), lambda i: (0, 0)),
            pl.BlockSpec((8, dh), lambda i: (0, 0)),
            pl.BlockSpec((1, 1), lambda i: (0, 0)),
            pl.BlockSpec((1, 1), lambda i: (0, 0)),
        ],
        out_specs=pl.BlockSpec((1, 1, 128), lambda i: (i, 0, 0)),
        out_shape=jax.ShapeDtypeStruct((b, 1, 128), jnp.float32),
        compiler_params=pltpu.CompilerParams(
            dimension_semantics=("parallel",)),
    )(x, x, x, x, W_enh.T.astype(jnp.bfloat16), b_enh.reshape(dh, 1),
      Wc1.T, Wc2.T, wstack, bt.reshape(1, 1), bcls.reshape(1, 1))
    return out[:, 0, :2 * k + 4]


# speedup vs baseline: 1.1925x; 1.1925x over previous
import jax
import jax.numpy as jnp
from jax.experimental import pallas as pl
from jax.experimental.pallas import tpu as pltpu

_CPB = 10


def _wsad_body(x1_ref, x2_ref, x3_ref, x4_ref, wet_ref, be_ref, wc1t_ref,
               wc2t_ref, wstack_ref, bt_ref, bcls_ref, out_ref,
               acc_feat, acc4):
    xrefs = (x1_ref, x2_ref, x3_ref, x4_ref)
    cpb = x1_ref.shape[1]
    t = x1_ref.shape[2]
    dq = x1_ref.shape[3]
    tt = cpb * t
    j = pl.program_id(1)
    nn = pl.num_programs(1)

    @pl.when(j == 0)
    def _init():
        acc_feat[...] = jnp.zeros_like(acc_feat)
        acc4[...] = jnp.zeros_like(acc4)

    parts = []
    for q, xr in enumerate(xrefs):
        xb = xr[0].reshape(tt, dq).astype(jnp.bfloat16)
        parts.append(jax.lax.dot_general(
            wet_ref[:, q * dq:(q + 1) * dq], xb, (((1,), (1,)), ((), ())),
            preferred_element_type=jnp.float32))
    hT = (parts[0] + parts[1]) + (parts[2] + parts[3])
    hT = jnp.maximum(hT + be_ref[...], 0.0)

    u = jax.lax.dot_general(
        wc1t_ref[...], hT, (((1,), (0,)), ((), ())),
        preferred_element_type=jnp.float32)
    ones8 = jnp.full((t, 8), 1.0 / t, jnp.float32)
    he_parts = []
    for c in range(cpb):
        g = jax.lax.dot_general(
            u[:, c * t:(c + 1) * t], ones8, (((1,), (0,)), ((), ())),
            preferred_element_type=jnp.float32)
        c1 = jnp.maximum(g, 0.0)
        c8 = jax.lax.dot_general(
            wc2t_ref[...], c1, (((1,), (0,)), ((), ())),
            preferred_element_type=jnp.float32)
        catten = jax.nn.sigmoid(c8[:, 0:1])
        he_parts.append(hT[:, c * t:(c + 1) * t] * catten)
    heT = jnp.concatenate(he_parts, axis=1)

    acc_feat[...] += sum(he_parts[1:], he_parts[0])

    z = jax.lax.dot_general(
        wstack_ref[...], hT, (((1,), (0,)), ((), ())),
        preferred_element_type=jnp.float32)
    zhe = jax.lax.dot_general(
        wstack_ref[...], heT, (((1,), (0,)), ((), ())),
        preferred_element_type=jnp.float32)

    tatt = jax.nn.sigmoid(z[0:1, :] + bt_ref[0, 0])
    score_e = jax.nn.sigmoid(zhe[1:2, :] + bcls_ref[0, 0])
    score_s = jax.nn.sigmoid(z[1:2, :] - zhe[1:2, :] + bcls_ref[0, 0])

    def crop_sum(v):
        r = v[:, 0:t]
        for c in range(1, cpb):
            r = r + v[:, c * t:(c + 1) * t]
        return r

    acc4[...] += jnp.concatenate(
        [crop_sum(score_e), crop_sum(score_s), crop_sum(tatt),
         crop_sum(1.0 - tatt), jnp.zeros((4, t), jnp.float32)], axis=0)

    @pl.when(j == nn - 1)
    def _fin():
        k = t // 16 + 1
        inv_n = 1.0 / (nn * cpb)
        a = acc4[...]
        score_e_m = a[0:1, :] * inv_n
        score_s_m = a[1:2, :] * inv_n
        te = a[2:3, :] * inv_n
        ts = a[3:4, :] * inv_n

        def softmax_row(v):
            e = jnp.exp(v - jnp.max(v))
            return e / jnp.sum(e)

        we_ = softmax_row(te)
        ws_ = softmax_row(ts)
        bag_ee = jnp.sum(score_e_m * we_)
        bag_es = jnp.sum(score_e_m * ws_)
        bag_se = jnp.sum(score_s_m * we_)
        bag_ss = jnp.sum(score_s_m * ws_)

        sc_scaled = score_e_m * te
        fm = acc_feat[...] * inv_n
        magsq = jnp.sum(fm * fm, axis=0, keepdims=True)
        rm = jnp.sqrt(magsq) * sc_scaled

        ir = jax.lax.broadcasted_iota(jnp.int32, (t, t), 0)
        ic = jax.lax.broadcasted_iota(jnp.int32, (t, t), 1)
        idm = (ir == ic).astype(jnp.float32)
        rm_col = jnp.transpose(rm)
        beats = (rm_col > rm) | ((rm_col == rm) & (ir < ic))
        m = beats.astype(jnp.float32)
        ones8r = jnp.ones((8, t), jnp.float32)
        rank8 = jax.lax.dot_general(
            ones8r, m, (((1,), (0,)), ((), ())),
            preferred_element_type=jnp.float32)
        rank_col = jax.lax.dot_general(
            idm, rank8, (((1,), (1,)), ((), ())),
            preferred_element_type=jnp.float32)[:, 0:1]
        ranks128 = jax.lax.broadcasted_iota(
            jnp.int32, (t, 128), 1).astype(jnp.float32)
        onehot = (rank_col == ranks128).astype(jnp.float32)
        v_sc = jax.lax.dot_general(
            sc_scaled, onehot, (((1,), (0,)), ((), ())),
            preferred_element_type=jnp.float32)
        v_ref = jax.lax.dot_general(
            rm, onehot, (((1,), (0,)), ((), ())),
            preferred_element_type=jnp.float32)

        row = jnp.concatenate([
            v_sc[:, :k], v_ref[:, :k],
            jnp.stack([bag_ee, bag_es, bag_se, bag_ss])[None, :],
            jnp.zeros((1, 128 - (2 * k + 4)), jnp.float32),
        ], axis=1)
        out_ref[0] = row


@jax.jit
def kernel(x, W_enh, b_enh, Wc1, Wc2, Wt, bt, Wcls, bcls):
    b, n, t, d = x.shape
    dh = W_enh.shape[1]
    dm = Wc1.shape[1]
    k = t // 16 + 1
    cpb = _CPB if n % _CPB == 0 else 1

    wstack = jnp.concatenate(
        [Wt.reshape(1, dh), Wcls.reshape(1, dh),
         jnp.zeros((6, dh), jnp.float32)], axis=0)

    out = pl.pallas_call(
        _wsad_body,
        grid=(b, n // cpb),
        in_specs=[
            pl.BlockSpec((1, cpb, t, d // 4), lambda i, j: (i, j, 0, 0)),
            pl.BlockSpec((1, cpb, t, d // 4), lambda i, j: (i, j, 0, 1)),
            pl.BlockSpec((1, cpb, t, d // 4), lambda i, j: (i, j, 0, 2)),
            pl.BlockSpec((1, cpb, t, d // 4), lambda i, j: (i, j, 0, 3)),
            pl.BlockSpec((dh, d), lambda i, j: (0, 0)),
            pl.BlockSpec((dh, 1), lambda i, j: (0, 0)),
            pl.BlockSpec((dm, dh), lambda i, j: (0, 0)),
            pl.BlockSpec((dh, dm), lambda i, j: (0, 0)),
            pl.BlockSpec((8, dh), lambda i, j: (0, 0)),
            pl.BlockSpec((1, 1), lambda i, j: (0, 0)),
            pl.BlockSpec((1, 1), lambda i, j: (0, 0)),
        ],
        out_specs=pl.BlockSpec((1, 1, 128), lambda i, j: (i, 0, 0)),
        out_shape=jax.ShapeDtypeStruct((b, 1, 128), jnp.float32),
        scratch_shapes=[
            pltpu.VMEM((dh, t), jnp.float32),
            pltpu.VMEM((8, t), jnp.float32),
        ],
        compiler_params=pltpu.CompilerParams(
            dimension_semantics=("parallel", "arbitrary")),
    )(x, x, x, x, W_enh.T.astype(jnp.bfloat16), b_enh.reshape(dh, 1),
      Wc1.T, Wc2.T, wstack, bt.reshape(1, 1), bcls.reshape(1, 1))
    return out[:, 0, :2 * k + 4]
